# baseline (speedup 1.0000x reference)
import functools

import jax
import jax.numpy as jnp
from jax import lax
from jax.experimental import pallas as pl
from jax.experimental.pallas import tpu as pltpu
from jax.experimental.pallas import tpu_sc as plsc

_IN = 2048
_OUT = 8192
_RANK = 2
_B = 4096
_K = _RANK * _OUT
_NC = 2
_NS = 16
_NW = _NC * _NS
_RW = _B // _NW
_L = 16
_R = 8
_G = _RW // _R
_SEG = 4096
_NSEG = _K // _SEG
_SPR = _OUT // _SEG


@functools.partial(
    pl.kernel,
    out_type=jax.ShapeDtypeStruct((_B, _RANK, _OUT), jnp.float32),
    mesh=plsc.VectorSubcoreMesh(core_axis_name="c", subcore_axis_name="s",
                                num_cores=_NC),
    scratch_types=[
        pltpu.VMEM((_K,), jnp.int32),
        pltpu.VMEM((_R * _IN,), jnp.float32),
        pltpu.VMEM((_R * _IN,), jnp.float32),
        pltpu.VMEM((_R * _SEG,), jnp.float32),
        pltpu.VMEM((_R * _SEG,), jnp.float32),
        pltpu.SemaphoreType.DMA,
        pltpu.SemaphoreType.DMA,
        pltpu.SemaphoreType.DMA,
        pltpu.SemaphoreType.DMA,
    ],
    compiler_params=pltpu.CompilerParams(needs_layout_passes=False),
)
def _lut_gather(x_hbm, idx_hbm, out_hbm, idx_v, x0, x1, o0, o1,
                si0, si1, so0, so1):
    wid = lax.axis_index("s") * _NC + lax.axis_index("c")
    base = wid * _RW
    for r in range(_RANK):
        pltpu.sync_copy(idx_hbm.at[r], idx_v.at[pl.ds(r * _OUT, _OUT)])

    xb, ob, si, so = (x0, x1), (o0, o1), (si0, si1), (so0, so1)

    def in_copies(g, buf, sem):
        row0 = base + g * _R
        return [
            pltpu.make_async_copy(
                x_hbm.at[row0 + q], buf.at[pl.ds(q * _IN, _IN)], sem)
            for q in range(_R)
        ]

    def seg_copies(g, s, buf, sem):
        row0 = base + g * _R
        return [
            pltpu.make_async_copy(
                buf.at[pl.ds(q * _SEG, _SEG)],
                out_hbm.at[row0 + q, s // _SPR,
                           pl.ds((s % _SPR) * _SEG, _SEG)],
                sem)
            for q in range(_R)
        ]

    for c in in_copies(0, x0, si0):
        c.start()

    def do_group(g, carry):
        gpar = lax.rem(g, 2)

        def run_parity(par):
            xv = xb[par]
            for c in in_copies(g, xv, si[par]):
                c.wait()

            @pl.when(g + 1 < _G)
            def _start_next():
                for c in in_copies(g + 1, xb[1 - par], si[1 - par]):
                    c.start()

            for s in range(_NSEG):
                sp = s % 2
                ov = ob[sp]
                if s >= 2:
                    for c in seg_copies(g, s, ov, so[sp]):
                        c.wait()
                else:
                    @pl.when(g >= 1)
                    def _wait_prev_group():
                        for c in seg_copies(g, s, ov, so[sp]):
                            c.wait()

                @plsc.parallel_loop(0, _SEG, step=_L, unroll=16)
                def chunk(off):
                    iv = idx_v[pl.ds(s * _SEG + off, _L)]
                    for q in range(_R):
                        ov[pl.ds(q * _SEG + off, _L)] = (
                            plsc.load_gather(xv, [iv + q * _IN]))

                for c in seg_copies(g, s, ov, so[sp]):
                    c.start()

        @pl.when(gpar == 0)
        def _p0():
            run_parity(0)

        @pl.when(gpar == 1)
        def _p1():
            run_parity(1)

        return carry

    lax.fori_loop(0, _G, do_group, 0)
    for c in seg_copies(_G - 1, _NSEG - 2, o0, so0):
        c.wait()
    for c in seg_copies(_G - 1, _NSEG - 1, o1, so1):
        c.wait()


@jax.jit
def kernel(x, indices):
    return _lut_gather(x, indices.astype(jnp.int32))


# speedup vs baseline: 1.4495x; 1.4495x over previous
import functools

import jax
import jax.numpy as jnp
from jax import lax
from jax.experimental import pallas as pl
from jax.experimental.pallas import tpu as pltpu
from jax.experimental.pallas import tpu_sc as plsc

_IN = 2048
_OUT = 8192
_RANK = 2
_B = 4096
_K = _RANK * _OUT
_NC = 2
_NS = 16
_NW = _NC * _NS
_RW = _B // _NW
_L = 16
_R = 8
_G = _RW // _R
_SEG = 4096
_NSEG = _K // _SEG
_SPR = _OUT // _SEG


@functools.partial(
    pl.kernel,
    out_type=jax.ShapeDtypeStruct((_B, _RANK, _OUT), jnp.float32),
    mesh=plsc.VectorSubcoreMesh(core_axis_name="c", subcore_axis_name="s",
                                num_cores=_NC),
    scratch_types=[
        pltpu.VMEM((_K,), jnp.int32),
        pltpu.VMEM((_R * _IN,), jnp.float32),
        pltpu.VMEM((_R * _IN,), jnp.float32),
        pltpu.VMEM((_R * _SEG,), jnp.float32),
        pltpu.VMEM((_R * _SEG,), jnp.float32),
        pltpu.SemaphoreType.DMA,
        pltpu.SemaphoreType.DMA,
        pltpu.SemaphoreType.DMA,
        pltpu.SemaphoreType.DMA,
    ],
    compiler_params=pltpu.CompilerParams(needs_layout_passes=False),
)
def _lut_gather(x_hbm, idx_hbm, out_hbm, idx_v, x0, x1, o0, o1,
                si0, si1, so0, so1):
    wid = lax.axis_index("s") * _NC + lax.axis_index("c")
    base = wid * _RW
    for r in range(_RANK):
        pltpu.sync_copy(idx_hbm.at[r], idx_v.at[pl.ds(r * _OUT, _OUT)])

    xb, ob, si, so = (x0, x1), (o0, o1), (si0, si1), (so0, so1)

    def in_copies(g, buf, sem):
        row0 = base + g * _R
        return [
            pltpu.make_async_copy(
                x_hbm.at[row0 + q], buf.at[pl.ds(q * _IN, _IN)], sem)
            for q in range(_R)
        ]

    def seg_copies(g, s, buf, sem):
        row0 = base + g * _R
        return [
            pltpu.make_async_copy(
                buf.at[pl.ds(q * _SEG, _SEG)],
                out_hbm.at[row0 + q, s // _SPR,
                           pl.ds((s % _SPR) * _SEG, _SEG)],
                sem)
            for q in range(_R)
        ]

    for c in in_copies(0, x0, si0):
        c.start()

    def do_group(g, carry):
        gpar = lax.rem(g, 2)

        def run_parity(par):
            xv = xb[par]
            for c in in_copies(g, xv, si[par]):
                c.wait()

            @pl.when(g + 1 < _G)
            def _start_next():
                for c in in_copies(g + 1, xb[1 - par], si[1 - par]):
                    c.start()

            for s in range(_NSEG):
                sp = s % 2
                ov = ob[sp]
                if s >= 2:
                    for c in seg_copies(g, s, ov, so[sp]):
                        c.wait()
                else:
                    @pl.when(g >= 1)
                    def _wait_prev_group():
                        for c in seg_copies(g, s, ov, so[sp]):
                            c.wait()

                @plsc.parallel_loop(0, _SEG, step=_L, unroll=8)
                def chunk(off):
                    iv = idx_v[pl.ds(s * _SEG + off, _L)]
                    for q in range(_R):
                        ov[pl.ds(q * _SEG + off, _L)] = (
                            plsc.load_gather(xv, [iv + q * _IN]))

                for c in seg_copies(g, s, ov, so[sp]):
                    c.start()

        @pl.when(gpar == 0)
        def _p0():
            run_parity(0)

        @pl.when(gpar == 1)
        def _p1():
            run_parity(1)

        return carry

    lax.fori_loop(0, _G, do_group, 0)
    for c in seg_copies(_G - 1, _NSEG - 2, o0, so0):
        c.wait()
    for c in seg_copies(_G - 1, _NSEG - 1, o1, so1):
        c.wait()


@jax.jit
def kernel(x, indices):
    return _lut_gather(x, indices.astype(jnp.int32))


# speedup vs baseline: 1.4609x; 1.0078x over previous
import functools

import jax
import jax.numpy as jnp
from jax import lax
from jax.experimental import pallas as pl
from jax.experimental.pallas import tpu as pltpu
from jax.experimental.pallas import tpu_sc as plsc

_IN = 2048
_OUT = 8192
_RANK = 2
_B = 4096
_K = _RANK * _OUT
_NC = 2
_NS = 16
_NW = _NC * _NS
_RW = _B // _NW
_L = 16
_R = 8
_G = _RW // _R
_SEG = 4096
_NSEG = _K // _SEG
_SPR = _OUT // _SEG


@functools.partial(
    pl.kernel,
    out_type=jax.ShapeDtypeStruct((_B, _RANK, _OUT), jnp.float32),
    mesh=plsc.VectorSubcoreMesh(core_axis_name="c", subcore_axis_name="s",
                                num_cores=_NC),
    scratch_types=[
        pltpu.VMEM((_K,), jnp.int32),
        pltpu.VMEM((_R * _IN,), jnp.float32),
        pltpu.VMEM((_R * _IN,), jnp.float32),
        pltpu.VMEM((_R * _SEG,), jnp.float32),
        pltpu.VMEM((_R * _SEG,), jnp.float32),
        pltpu.SemaphoreType.DMA,
        pltpu.SemaphoreType.DMA,
        pltpu.SemaphoreType.DMA,
        pltpu.SemaphoreType.DMA,
    ],
    compiler_params=pltpu.CompilerParams(needs_layout_passes=False),
)
def _lut_gather(x_hbm, idx_hbm, out_hbm, idx_v, x0, x1, o0, o1,
                si0, si1, so0, so1):
    wid = lax.axis_index("s") * _NC + lax.axis_index("c")
    base = wid * _RW

    xb, ob, si, so = (x0, x1), (o0, o1), (si0, si1), (so0, so1)

    def in_copies(g, buf, sem):
        row0 = base + g * _R
        return [
            pltpu.make_async_copy(
                x_hbm.at[row0 + q], buf.at[pl.ds(q * _IN, _IN)], sem)
            for q in range(_R)
        ]

    def seg_copies(g, s, buf, sem):
        row0 = base + g * _R
        return [
            pltpu.make_async_copy(
                buf.at[pl.ds(q * _SEG, _SEG)],
                out_hbm.at[row0 + q, s // _SPR,
                           pl.ds((s % _SPR) * _SEG, _SEG)],
                sem)
            for q in range(_R)
        ]

    for c in in_copies(0, x0, si0):
        c.start()
    for r in range(_RANK):
        pltpu.sync_copy(idx_hbm.at[r], idx_v.at[pl.ds(r * _OUT, _OUT)])

    def do_group(g, carry):
        gpar = lax.rem(g, 2)

        def run_parity(par):
            xv = xb[par]
            for c in in_copies(g, xv, si[par]):
                c.wait()

            @pl.when(g + 1 < _G)
            def _start_next():
                for c in in_copies(g + 1, xb[1 - par], si[1 - par]):
                    c.start()

            for s in range(_NSEG):
                sp = s % 2
                ov = ob[sp]
                if s >= 2:
                    for c in seg_copies(g, s, ov, so[sp]):
                        c.wait()
                else:
                    @pl.when(g >= 1)
                    def _wait_prev_group():
                        for c in seg_copies(g, s, ov, so[sp]):
                            c.wait()

                @plsc.parallel_loop(0, _SEG, step=_L, unroll=8)
                def chunk(off):
                    iv = idx_v[pl.ds(s * _SEG + off, _L)]
                    for q in range(_R):
                        ov[pl.ds(q * _SEG + off, _L)] = (
                            plsc.load_gather(xv, [iv + q * _IN]))

                for c in seg_copies(g, s, ov, so[sp]):
                    c.start()

        @pl.when(gpar == 0)
        def _p0():
            run_parity(0)

        @pl.when(gpar == 1)
        def _p1():
            run_parity(1)

        return carry

    lax.fori_loop(0, _G, do_group, 0)
    for c in seg_copies(_G - 1, _NSEG - 2, o0, so0):
        c.wait()
    for c in seg_copies(_G - 1, _NSEG - 1, o1, so1):
        c.wait()


@jax.jit
def kernel(x, indices):
    return _lut_gather(x, indices.astype(jnp.int32))


# speedup vs baseline: 1.4837x; 1.0156x over previous
import functools

import jax
import jax.numpy as jnp
from jax import lax
from jax.experimental import pallas as pl
from jax.experimental.pallas import tpu as pltpu
from jax.experimental.pallas import tpu_sc as plsc

_IN = 2048
_OUT = 8192
_RANK = 2
_B = 4096
_K = _RANK * _OUT
_NC = 2
_NS = 16
_NW = _NC * _NS
_RW = _B // _NW
_L = 16
_R = 8
_G = _RW // _R
_SEG = 4096
_NSEG = _K // _SEG
_SPR = _OUT // _SEG


@functools.partial(
    pl.kernel,
    out_type=jax.ShapeDtypeStruct((_B, _RANK, _OUT), jnp.float32),
    mesh=plsc.VectorSubcoreMesh(core_axis_name="c", subcore_axis_name="s",
                                num_cores=_NC),
    scratch_types=[
        pltpu.VMEM((_K,), jnp.int32),
        pltpu.VMEM((_R * _IN,), jnp.float32),
        pltpu.VMEM((_R * _IN,), jnp.float32),
        pltpu.VMEM((_R * _SEG,), jnp.float32),
        pltpu.VMEM((_R * _SEG,), jnp.float32),
        pltpu.SemaphoreType.DMA,
        pltpu.SemaphoreType.DMA,
        pltpu.SemaphoreType.DMA,
        pltpu.SemaphoreType.DMA,
    ],
    compiler_params=pltpu.CompilerParams(needs_layout_passes=False),
)
def _lut_gather(x_hbm, idx_hbm, out_hbm, idx_v, x0, x1, o0, o1,
                si0, si1, so0, so1):
    wid = lax.axis_index("s") * _NC + lax.axis_index("c")
    base = wid * _RW

    xb, ob, si, so = (x0, x1), (o0, o1), (si0, si1), (so0, so1)

    def in_copies(g, buf, sem):
        row0 = base + g * _R
        return [
            pltpu.make_async_copy(
                x_hbm.at[row0 + q], buf.at[pl.ds(q * _IN, _IN)], sem)
            for q in range(_R)
        ]

    def seg_copies(g, s, buf, sem):
        row0 = base + g * _R
        return [
            pltpu.make_async_copy(
                buf.at[pl.ds(q * _SEG, _SEG)],
                out_hbm.at[row0 + q, s // _SPR,
                           pl.ds((s % _SPR) * _SEG, _SEG)],
                sem)
            for q in range(_R)
        ]

    for c in in_copies(0, x0, si0):
        c.start()
    for r in range(_RANK):
        pltpu.sync_copy(idx_hbm.at[r], idx_v.at[pl.ds(r * _OUT, _OUT)])

    def do_group(g, carry):
        gpar = lax.rem(g, 2)

        def run_parity(par):
            xv = xb[par]
            for c in in_copies(g, xv, si[par]):
                c.wait()

            @pl.when(g + 1 < _G)
            def _start_next():
                for c in in_copies(g + 1, xb[1 - par], si[1 - par]):
                    c.start()

            def seg_pair(spair, c2):
                for sp in range(2):
                    s = spair * 2 + sp
                    ov = ob[sp]

                    @pl.when((g >= 1) | (spair >= 1))
                    def _wait_out():
                        for c in seg_copies(g, s, ov, so[sp]):
                            c.wait()

                    @plsc.parallel_loop(0, _SEG, step=_L, unroll=8)
                    def chunk(off):
                        iv = idx_v[pl.ds(s * _SEG + off, _L)]
                        for q in range(_R):
                            ov[pl.ds(q * _SEG + off, _L)] = (
                                plsc.load_gather(xv, [iv + q * _IN]))

                    for c in seg_copies(g, s, ov, so[sp]):
                        c.start()
                return c2

            lax.fori_loop(0, _NSEG // 2, seg_pair, 0)

        @pl.when(gpar == 0)
        def _p0():
            run_parity(0)

        @pl.when(gpar == 1)
        def _p1():
            run_parity(1)

        return carry

    lax.fori_loop(0, _G, do_group, 0)
    for c in seg_copies(_G - 1, _NSEG - 2, o0, so0):
        c.wait()
    for c in seg_copies(_G - 1, _NSEG - 1, o1, so1):
        c.wait()


@jax.jit
def kernel(x, indices):
    return _lut_gather(x, indices.astype(jnp.int32))
